# baseline (speedup 1.0000x reference)
import functools

import jax
import jax.numpy as jnp
from jax import lax
from jax.experimental import pallas as pl
from jax.experimental.pallas import tpu as pltpu
from jax.experimental.pallas import tpu_sc as plsc

_MARGIN_MIN = 0.2
_MARGIN_MAX = 0.5
_B = 4096
_D = 256
_RB = 512
_NBLK = _B // _RB

_CH = 128
_NCH = _B // _CH


def _mine_body(u_ref, tu_ref, vub_ref, vu_ref, e_ref, f_ref,
               tc_ref, vc_ref, w1t_ref, w1v_ref, w1d_ref, b1_ref,
               w2_ref, b2_ref, idx_ref, pos_ref):
    i = pl.program_id(0)
    tu = tu_ref[...]
    vu = vu_ref[...]
    S = lax.dot_general(tu, vu, (((1,), (1,)), ((), ())),
                        preferred_element_type=jnp.float32)
    rows = i * _RB + lax.broadcasted_iota(jnp.int32, (_RB, _B), 0)
    cols = lax.broadcasted_iota(jnp.int32, (_RB, _B), 1)
    on_diag = rows == cols
    diag = jnp.sum(tu * vub_ref[...], axis=1, keepdims=True)
    band = (S > diag - _MARGIN_MAX) & (S < diag - _MARGIN_MIN)
    bf = band.astype(jnp.bfloat16)
    p = jnp.dot(bf, e_ref[...],
                preferred_element_type=jnp.float32)
    count = p[:, _NCH - 1:_NCH]
    u = u_ref[...]
    k = jnp.floor(u * jnp.maximum(count, 1.0))
    p_excl = jnp.concatenate(
        [jnp.zeros((_RB, 1), jnp.float32), p[:, :_NCH - 1]], axis=1)
    in_chunk = (p_excl <= k) & (k < p)
    ch_iota = lax.broadcasted_iota(jnp.int32, (_RB, _NCH), 1)
    c_star = jnp.sum(jnp.where(in_chunk, ch_iota, 0), axis=1, keepdims=True)
    r = k - jnp.sum(jnp.where(in_chunk, p_excl, 0.0), axis=1, keepdims=True)
    sel_chunk = lax.shift_right_logical(cols, 7) == c_star
    masked = (sel_chunk & band).astype(jnp.bfloat16)
    q = jnp.dot(masked, f_ref[...],
                preferred_element_type=jnp.float32)
    lane_iota = lax.broadcasted_iota(jnp.int32, (_RB, _CH), 1)
    pos = jnp.min(jnp.where(q == r + 1.0, lane_iota, _CH),
                  axis=1, keepdims=True).astype(jnp.float32)
    cand = c_star.astype(jnp.float32) * float(_CH) + pos
    colsf = cols.astype(jnp.float32)
    s_masked = jnp.where(on_diag, -3e38, S)
    m = jnp.max(s_masked, axis=1, keepdims=True)
    fb = jnp.min(jnp.where(s_masked == m, colsf, float(_B)), axis=1, keepdims=True)
    neg = jnp.where(count > 0.0, cand, fb)
    idx_ref[...] = neg.astype(jnp.int32)
    tc = tc_ref[...]
    vc = vc_ref[...]
    dot_pos = jnp.sum(vc * tc, axis=1, keepdims=True)
    h_pos = (jnp.dot(tc, w1t_ref[...], preferred_element_type=jnp.float32)
             + jnp.dot(vc, w1v_ref[...], preferred_element_type=jnp.float32)
             + dot_pos * w1d_ref[...] + b1_ref[...])
    h_pos = jnp.maximum(h_pos, 0.0)
    lp = jnp.sum(h_pos * w2_ref[...], axis=1, keepdims=True) + b2_ref[...]
    pos_part = jnp.sum(jnp.log(jax.nn.sigmoid(lp) + 1e-08)).reshape(1, 1)

    @pl.when(i == 0)
    def _():
        pos_ref[...] = jnp.zeros((1, 1), jnp.float32)

    pos_ref[...] += pos_part


def _mine_and_pos(text_uni, vision_uni, u_col, e_mat, f_mat,
                  tc, vc, w1t, w1v, w1d, b1, w2, b2):
    blk = lambda r, c: pl.BlockSpec((r, c), lambda i: (i, 0))
    full = lambda r, c: pl.BlockSpec((r, c), lambda i: (0, 0))
    return pl.pallas_call(
        _mine_body,
        grid=(_NBLK,),
        in_specs=[
            blk(_RB, 1), blk(_RB, _D), blk(_RB, _D), full(_B, _D),
            full(_B, _NCH), full(_B, _CH),
            blk(_RB, _D), blk(_RB, _D), full(_D, _D), full(_D, _D),
            full(1, _D), full(1, _D), full(1, _D), full(1, 1),
        ],
        out_specs=[blk(_RB, 1), full(1, 1)],
        out_shape=[
            jax.ShapeDtypeStruct((_B, 1), jnp.int32),
            jax.ShapeDtypeStruct((1, 1), jnp.float32),
        ],
    )(u_col, text_uni, vision_uni, vision_uni, e_mat, f_mat,
      tc, vc, w1t, w1v, w1d, b1, w2, b2)


def _sc_gather(table, idx):
    info = plsc.get_sparse_core_info()
    nw = info.num_cores * info.num_subcores
    b_per_w = _B // nw
    mesh = plsc.VectorSubcoreMesh(core_axis_name="c", subcore_axis_name="s")

    @functools.partial(
        pl.kernel,
        mesh=mesh,
        out_type=jax.ShapeDtypeStruct((_B, _D), jnp.float32),
        scratch_types=[
            pltpu.VMEM((b_per_w,), jnp.int32),
            pltpu.VMEM((b_per_w, _D), jnp.float32),
            pltpu.SemaphoreType.DMA,
        ],
    )
    def gk(table_hbm, idx_hbm, out_hbm, idx_v, rows_v, sem):
        wid = lax.axis_index("s") * info.num_cores + lax.axis_index("c")
        base = wid * b_per_w
        pltpu.sync_copy(idx_hbm.at[pl.ds(base, b_per_w)], idx_v)
        pltpu.async_copy(table_hbm.at[idx_v], rows_v, sem).wait()
        pltpu.sync_copy(rows_v, out_hbm.at[pl.ds(base, b_per_w)])

    return gk(table, idx)


def _neg_body(tc_ref, vn_ref, w1t_ref, w1v_ref, w1d_ref, b1_ref,
              w2_ref, b2_ref, pos_ref, loss_ref):
    i = pl.program_id(0)
    tc = tc_ref[...]
    vn = vn_ref[...]
    dot_neg = jnp.sum(vn * tc, axis=1, keepdims=True)
    h_neg = (jnp.dot(tc, w1t_ref[...], preferred_element_type=jnp.float32)
             + jnp.dot(vn, w1v_ref[...], preferred_element_type=jnp.float32)
             + dot_neg * w1d_ref[...] + b1_ref[...])
    h_neg = jnp.maximum(h_neg, 0.0)
    ln = jnp.sum(h_neg * w2_ref[...], axis=1, keepdims=True) + b2_ref[...]
    neg_part = jnp.sum(jnp.log(1.0 - jax.nn.sigmoid(ln) + 1e-08)).reshape(1, 1)

    @pl.when(i == 0)
    def _():
        loss_ref[...] = jnp.zeros((1, 1), jnp.float32)

    loss_ref[...] += neg_part

    @pl.when(i == _NBLK - 1)
    def _():
        ns = loss_ref[...]
        ps = pos_ref[...]
        loss_ref[...] = ((-ps / _B) + (-ns / _B)) * 0.5


def _neg_loss(tc, vn, w1t, w1v, w1d, b1, w2, b2, pos_sum):
    blk = lambda r, c: pl.BlockSpec((r, c), lambda i: (i, 0))
    full = lambda r, c: pl.BlockSpec((r, c), lambda i: (0, 0))
    return pl.pallas_call(
        _neg_body,
        grid=(_NBLK,),
        in_specs=[
            blk(_RB, _D), blk(_RB, _D), full(_D, _D), full(_D, _D),
            full(1, _D), full(1, _D), full(1, _D), full(1, 1),
            full(1, 1),
        ],
        out_specs=full(1, 1),
        out_shape=jax.ShapeDtypeStruct((1, 1), jnp.float32),
    )(tc, vn, w1t, w1v, w1d, b1, w2, b2, pos_sum)


def kernel(vision_embeds_cross, text_embeds_cross, vision_embeds_uni,
           text_embeds_uni, W1, b1, W2, b2):
    u = jax.random.uniform(jax.random.key(42), (_B,))
    j = jnp.arange(_B)
    e_mat = (j[:, None] // _CH <= jnp.arange(_NCH)[None, :]).astype(jnp.bfloat16)
    f_mat = (j[:, None] % _CH <= jnp.arange(_CH)[None, :]).astype(jnp.bfloat16)
    w1t = W1[:_D]
    w1v = W1[_D:2 * _D]
    w1d = W1[2 * _D:2 * _D + 1]
    b1r = b1[None, :]
    w2r = W2.reshape(1, _D)
    b2r = b2[:, None]
    neg_col, pos_sum = _mine_and_pos(
        text_embeds_uni, vision_embeds_uni, u[:, None], e_mat, f_mat,
        text_embeds_cross, vision_embeds_cross, w1t, w1v, w1d, b1r, w2r, b2r)
    vision_neg = _sc_gather(vision_embeds_cross, neg_col[:, 0])
    loss = _neg_loss(text_embeds_cross, vision_neg,
                     w1t, w1v, w1d, b1r, w2r, b2r, pos_sum)
    return loss[0, 0]


# speedup vs baseline: 1.0143x; 1.0143x over previous
import functools

import jax
import jax.numpy as jnp
from jax import lax
from jax.experimental import pallas as pl
from jax.experimental.pallas import tpu as pltpu
from jax.experimental.pallas import tpu_sc as plsc

_MARGIN_MIN = 0.2
_MARGIN_MAX = 0.5
_B = 4096
_D = 256
_RB = 1024
_NBLK = _B // _RB

_CH = 128
_NCH = _B // _CH


def _mine_body(u_ref, tu_ref, vub_ref, vu_ref, e_ref, f_ref,
               tc_ref, vc_ref, w1t_ref, w1v_ref, w1d_ref, b1_ref,
               w2_ref, b2_ref, idx_ref, pos_ref):
    i = pl.program_id(0)
    tu = tu_ref[...]
    vu = vu_ref[...]
    S = lax.dot_general(tu, vu, (((1,), (1,)), ((), ())),
                        preferred_element_type=jnp.float32)
    rows = i * _RB + lax.broadcasted_iota(jnp.int32, (_RB, _B), 0)
    cols = lax.broadcasted_iota(jnp.int32, (_RB, _B), 1)
    on_diag = rows == cols
    diag = jnp.sum(tu * vub_ref[...], axis=1, keepdims=True)
    band = (S > diag - _MARGIN_MAX) & (S < diag - _MARGIN_MIN)
    bf = band.astype(jnp.bfloat16)
    p = jnp.dot(bf, e_ref[...],
                preferred_element_type=jnp.float32)
    count = p[:, _NCH - 1:_NCH]
    u = u_ref[...]
    k = jnp.floor(u * jnp.maximum(count, 1.0))
    p_excl = jnp.concatenate(
        [jnp.zeros((_RB, 1), jnp.float32), p[:, :_NCH - 1]], axis=1)
    in_chunk = (p_excl <= k) & (k < p)
    ch_iota = lax.broadcasted_iota(jnp.int32, (_RB, _NCH), 1)
    c_star = jnp.sum(jnp.where(in_chunk, ch_iota, 0), axis=1, keepdims=True)
    r = k - jnp.sum(jnp.where(in_chunk, p_excl, 0.0), axis=1, keepdims=True)
    sel_chunk = lax.shift_right_logical(cols, 7) == c_star
    masked = (sel_chunk & band).astype(jnp.bfloat16)
    q = jnp.dot(masked, f_ref[...],
                preferred_element_type=jnp.float32)
    lane_iota = lax.broadcasted_iota(jnp.int32, (_RB, _CH), 1)
    pos = jnp.min(jnp.where(q == r + 1.0, lane_iota, _CH),
                  axis=1, keepdims=True).astype(jnp.float32)
    cand = c_star.astype(jnp.float32) * float(_CH) + pos
    colsf = cols.astype(jnp.float32)
    s_masked = jnp.where(on_diag, -3e38, S)
    m = jnp.max(s_masked, axis=1, keepdims=True)
    fb = jnp.min(jnp.where(s_masked == m, colsf, float(_B)), axis=1, keepdims=True)
    neg = jnp.where(count > 0.0, cand, fb)
    idx_ref[...] = neg.astype(jnp.int32)
    tc = tc_ref[...]
    vc = vc_ref[...]
    dot_pos = jnp.sum(vc * tc, axis=1, keepdims=True)
    h_pos = (jnp.dot(tc, w1t_ref[...], preferred_element_type=jnp.float32)
             + jnp.dot(vc, w1v_ref[...], preferred_element_type=jnp.float32)
             + dot_pos * w1d_ref[...] + b1_ref[...])
    h_pos = jnp.maximum(h_pos, 0.0)
    lp = jnp.sum(h_pos * w2_ref[...], axis=1, keepdims=True) + b2_ref[...]
    pos_part = jnp.sum(jnp.log(jax.nn.sigmoid(lp) + 1e-08)).reshape(1, 1)

    @pl.when(i == 0)
    def _():
        pos_ref[...] = jnp.zeros((1, 1), jnp.float32)

    pos_ref[...] += pos_part


def _mine_and_pos(text_uni, vision_uni, u_col, e_mat, f_mat,
                  tc, vc, w1t, w1v, w1d, b1, w2, b2):
    blk = lambda r, c: pl.BlockSpec((r, c), lambda i: (i, 0))
    full = lambda r, c: pl.BlockSpec((r, c), lambda i: (0, 0))
    return pl.pallas_call(
        _mine_body,
        grid=(_NBLK,),
        in_specs=[
            blk(_RB, 1), blk(_RB, _D), blk(_RB, _D), full(_B, _D),
            full(_B, _NCH), full(_B, _CH),
            blk(_RB, _D), blk(_RB, _D), full(_D, _D), full(_D, _D),
            full(1, _D), full(1, _D), full(1, _D), full(1, 1),
        ],
        out_specs=[blk(_RB, 1), full(1, 1)],
        out_shape=[
            jax.ShapeDtypeStruct((_B, 1), jnp.int32),
            jax.ShapeDtypeStruct((1, 1), jnp.float32),
        ],
    )(u_col, text_uni, vision_uni, vision_uni, e_mat, f_mat,
      tc, vc, w1t, w1v, w1d, b1, w2, b2)


def _sc_gather(table, idx):
    info = plsc.get_sparse_core_info()
    nw = info.num_cores * info.num_subcores
    b_per_w = _B // nw
    mesh = plsc.VectorSubcoreMesh(core_axis_name="c", subcore_axis_name="s")

    @functools.partial(
        pl.kernel,
        mesh=mesh,
        out_type=jax.ShapeDtypeStruct((_B, _D), jnp.float32),
        scratch_types=[
            pltpu.VMEM((b_per_w,), jnp.int32),
            pltpu.VMEM((b_per_w, _D), jnp.float32),
            pltpu.SemaphoreType.DMA,
        ],
    )
    def gk(table_hbm, idx_hbm, out_hbm, idx_v, rows_v, sem):
        wid = lax.axis_index("s") * info.num_cores + lax.axis_index("c")
        base = wid * b_per_w
        pltpu.sync_copy(idx_hbm.at[pl.ds(base, b_per_w)], idx_v)
        pltpu.async_copy(table_hbm.at[idx_v], rows_v, sem).wait()
        pltpu.sync_copy(rows_v, out_hbm.at[pl.ds(base, b_per_w)])

    return gk(table, idx)


def _neg_body(tc_ref, vn_ref, w1t_ref, w1v_ref, w1d_ref, b1_ref,
              w2_ref, b2_ref, pos_ref, loss_ref):
    i = pl.program_id(0)
    tc = tc_ref[...]
    vn = vn_ref[...]
    dot_neg = jnp.sum(vn * tc, axis=1, keepdims=True)
    h_neg = (jnp.dot(tc, w1t_ref[...], preferred_element_type=jnp.float32)
             + jnp.dot(vn, w1v_ref[...], preferred_element_type=jnp.float32)
             + dot_neg * w1d_ref[...] + b1_ref[...])
    h_neg = jnp.maximum(h_neg, 0.0)
    ln = jnp.sum(h_neg * w2_ref[...], axis=1, keepdims=True) + b2_ref[...]
    neg_part = jnp.sum(jnp.log(1.0 - jax.nn.sigmoid(ln) + 1e-08)).reshape(1, 1)

    @pl.when(i == 0)
    def _():
        loss_ref[...] = jnp.zeros((1, 1), jnp.float32)

    loss_ref[...] += neg_part

    @pl.when(i == _NBLK - 1)
    def _():
        ns = loss_ref[...]
        ps = pos_ref[...]
        loss_ref[...] = ((-ps / _B) + (-ns / _B)) * 0.5


def _neg_loss(tc, vn, w1t, w1v, w1d, b1, w2, b2, pos_sum):
    blk = lambda r, c: pl.BlockSpec((r, c), lambda i: (i, 0))
    full = lambda r, c: pl.BlockSpec((r, c), lambda i: (0, 0))
    return pl.pallas_call(
        _neg_body,
        grid=(_NBLK,),
        in_specs=[
            blk(_RB, _D), blk(_RB, _D), full(_D, _D), full(_D, _D),
            full(1, _D), full(1, _D), full(1, _D), full(1, 1),
            full(1, 1),
        ],
        out_specs=full(1, 1),
        out_shape=jax.ShapeDtypeStruct((1, 1), jnp.float32),
    )(tc, vn, w1t, w1v, w1d, b1, w2, b2, pos_sum)


def kernel(vision_embeds_cross, text_embeds_cross, vision_embeds_uni,
           text_embeds_uni, W1, b1, W2, b2):
    u = jax.random.uniform(jax.random.key(42), (_B,))
    j = jnp.arange(_B)
    e_mat = (j[:, None] // _CH <= jnp.arange(_NCH)[None, :]).astype(jnp.bfloat16)
    f_mat = (j[:, None] % _CH <= jnp.arange(_CH)[None, :]).astype(jnp.bfloat16)
    w1t = W1[:_D]
    w1v = W1[_D:2 * _D]
    w1d = W1[2 * _D:2 * _D + 1]
    b1r = b1[None, :]
    w2r = W2.reshape(1, _D)
    b2r = b2[:, None]
    neg_col, pos_sum = _mine_and_pos(
        text_embeds_uni, vision_embeds_uni, u[:, None], e_mat, f_mat,
        text_embeds_cross, vision_embeds_cross, w1t, w1v, w1d, b1r, w2r, b2r)
    vision_neg = _sc_gather(vision_embeds_cross, neg_col[:, 0])
    loss = _neg_loss(text_embeds_cross, vision_neg,
                     w1t, w1v, w1d, b1r, w2r, b2r, pos_sum)
    return loss[0, 0]
